# baseline (speedup 1.0000x reference)
import functools

import jax
import jax.numpy as jnp
from jax import lax
from jax.experimental import pallas as pl
from jax.experimental.pallas import tpu as pltpu
from jax.experimental.pallas import tpu_sc as plsc

EMB = 128
N_EDGES = 320000
ROWS01 = 21
ROWS2 = 3
T_PAD = 128

NW = 32
B_W = N_EDGES // NW
CHUNK = 400
TRIPS = B_W // CHUNK
GCH = 100
NG = CHUNK // GCH


def _prep_body(ea_ref, w0_ref, w1_ref, w2_ref, c_ref, t_ref):
    c_ref[...] = ea_ref[0] * ROWS01 + ea_ref[1] * ROWS2 + ea_ref[2]
    r = lax.broadcasted_iota(jnp.int32, (T_PAD, 1), 0)
    i0 = r // ROWS01
    i1 = (r % ROWS01) // ROWS2
    i2 = r % ROWS2
    oh0 = (i0 == lax.broadcasted_iota(jnp.int32, (T_PAD, 6), 1)).astype(jnp.float32)
    oh1 = (i1 == lax.broadcasted_iota(jnp.int32, (T_PAD, 7), 1)).astype(jnp.float32)
    oh2 = (i2 == lax.broadcasted_iota(jnp.int32, (T_PAD, 3), 1)).astype(jnp.float32)
    t = jnp.dot(oh0, w0_ref[...], preferred_element_type=jnp.float32)
    t += jnp.dot(oh1, w1_ref[...], preferred_element_type=jnp.float32)
    t += jnp.dot(oh2, w2_ref[...], preferred_element_type=jnp.float32)
    t_ref[...] = t


def _sc_body(t_hbm, c_hbm, out_hbm, idx_v, ra, rb, gsem, ssem):
    bank = [ra, rb]
    wid = lax.axis_index("s") * 2 + lax.axis_index("c")
    pltpu.sync_copy(c_hbm.at[wid], idx_v)

    def gathers(t, dst):
        for j in range(NG):
            pltpu.async_copy(
                t_hbm.at[idx_v.at[t * NG + j]],
                dst.at[pl.ds(j * GCH, GCH)], gsem)

    def wait_gathers(t, dst):
        for j in range(NG):
            pltpu.make_async_copy(
                t_hbm.at[idx_v.at[t * NG + j]],
                dst.at[pl.ds(j * GCH, GCH)], gsem).wait()

    gathers(0, bank[0])

    def out_slc(t):
        return out_hbm.at[pl.ds(wid * B_W + t * CHUNK, CHUNK)]

    def trip(t, bk):
        rows, nrows = bank[bk], bank[1 - bk]
        wait_gathers(t, rows)

        @pl.when(t >= 1)
        def _():
            pltpu.make_async_copy(nrows, out_slc(t - 1), ssem).wait()

        @pl.when(t < TRIPS - 1)
        def _():
            gathers(t + 1, nrows)

        pltpu.async_copy(rows, out_slc(t), ssem)

    def dbl(p, carry):
        trip(2 * p, 0)
        trip(2 * p + 1, 1)
        return carry

    lax.fori_loop(0, (TRIPS - 1) // 2, dbl, 0)
    if TRIPS % 2:
        trip(TRIPS - 1, 0)
    pltpu.make_async_copy(bank[(TRIPS - 1) % 2], out_slc(TRIPS - 1), ssem).wait()


@jax.jit
def _run(ea_t, W0, W1, W2):
    c2d, table = pl.pallas_call(
        _prep_body,
        out_shape=(
            jax.ShapeDtypeStruct((N_EDGES // EMB, EMB), jnp.int32),
            jax.ShapeDtypeStruct((T_PAD, EMB), jnp.float32),
        ),
    )(ea_t, W0, W1, W2)

    mesh = plsc.VectorSubcoreMesh(core_axis_name="c", subcore_axis_name="s")
    sc = functools.partial(
        pl.kernel,
        out_type=jax.ShapeDtypeStruct((N_EDGES, EMB), jnp.float32),
        mesh=mesh,
        scratch_types=[
            pltpu.VMEM((TRIPS * NG, GCH), jnp.int32),
            pltpu.VMEM((CHUNK, EMB), jnp.float32),
            pltpu.VMEM((CHUNK, EMB), jnp.float32),
            pltpu.SemaphoreType.DMA,
            pltpu.SemaphoreType.DMA,
        ],
    )(_sc_body)
    return sc(table, c2d.reshape(NW, TRIPS * NG, GCH))


def kernel(edge_attr, W0, W1, W2):
    ea_t = edge_attr.astype(jnp.int32).T.reshape(3, N_EDGES // EMB, EMB)
    return _run(ea_t, W0, W1, W2)
